# Initial kernel scaffold; baseline (speedup 1.0000x reference)
import jax
import jax.numpy as jnp
from jax.experimental import pallas as pl


def kernel(path_length, bucket_embedding):
    raise NotImplementedError("write your pallas kernel here")


# speedup vs baseline: 9.6780x; 9.6780x over previous
import jax
import jax.numpy as jnp
from jax.experimental import pallas as pl

NUM_ROWS = 3
DIM = 256
ROWS_PER_BLOCK = 2048


def _expand_body(idx_ref, table_ref, out_ref):
    idx = idx_ref[0, 0, :]
    b = jnp.clip(idx - 1, 0, NUM_ROWS - 1)
    b2 = b[:, None]
    row0 = table_ref[0:1, :]
    row1 = table_ref[1:2, :]
    row2 = table_ref[2:3, :]
    out_ref[0] = jnp.where(b2 == 0, row0, jnp.where(b2 == 1, row1, row2))


def kernel(path_length, bucket_embedding):
    shape = path_length.shape
    n = path_length.size
    grid = n // ROWS_PER_BLOCK
    idx3 = path_length.reshape(grid, 1, ROWS_PER_BLOCK).astype(jnp.int32)
    out = pl.pallas_call(
        _expand_body,
        grid=(grid,),
        in_specs=[
            pl.BlockSpec((1, 1, ROWS_PER_BLOCK), lambda i: (i, 0, 0)),
            pl.BlockSpec((NUM_ROWS, DIM), lambda i: (0, 0)),
        ],
        out_specs=pl.BlockSpec((1, ROWS_PER_BLOCK, DIM), lambda i: (i, 0, 0)),
        out_shape=jax.ShapeDtypeStruct((grid, ROWS_PER_BLOCK, DIM), jnp.float32),
    )(idx3, bucket_embedding)
    return out.reshape(*shape, DIM)
